# baseline (speedup 1.0000x reference)
import functools

import jax
import jax.numpy as jnp
from jax import lax
from jax.experimental import pallas as pl
from jax.experimental.pallas import tpu as pltpu
from jax.experimental.pallas import tpu_sc as plsc

BATCH = 16384
EMBED_DIM = 64
SLAB = 512
CAP = 4224


def _excl_prefix(mi, iota):
    s = mi
    for sh in (1, 2, 4, 8):
        idx = jnp.maximum(iota - sh, 0)
        shifted = s.at[idx].get(mode="promise_in_bounds")
        s = s + jnp.where(iota >= sh, shifted, 0)
    return s - mi


def _make_sc_gather(num_users, num_items):
    info = plsc.get_sparse_core_info()
    nc, ns = info.num_cores, info.num_subcores
    nw = nc * ns

    u_nslab = num_users // SLAB
    u_main = u_nslab * SLAB
    u_tail = num_users - u_main
    i_nslab = num_items // SLAB
    i_main = i_nslab * SLAB
    i_tail = num_items - i_main
    u_bound = -(-u_nslab // nw)
    i_bound = -(-i_nslab // nw)

    mesh = plsc.VectorSubcoreMesh(core_axis_name="c", subcore_axis_name="s")

    @functools.partial(
        pl.kernel,
        mesh=mesh,
        out_type=[
            jax.ShapeDtypeStruct((BATCH, 128), jnp.float32),
            jax.ShapeDtypeStruct((BATCH, 128), jnp.float32),
        ],
        scratch_types=[
            pltpu.VMEM((BATCH,), jnp.int32),
            pltpu.VMEM((CAP + 16,), jnp.int32),
            pltpu.VMEM((CAP + 16,), jnp.int32),
            pltpu.VMEM((CAP + 16,), jnp.int32),
            pltpu.VMEM((CAP + 16,), jnp.int32),
            pltpu.VMEM((EMBED_DIM, SLAB), jnp.float32),
            pltpu.VMEM((EMBED_DIM, SLAB), jnp.float32),
            pltpu.VMEM((EMBED_DIM, u_tail), jnp.float32),
            pltpu.VMEM((EMBED_DIM, i_tail), jnp.float32),
            pltpu.VMEM((16, 128), jnp.float32),
            pltpu.VMEM((16, 128), jnp.float32),
            pltpu.SemaphoreType.DMA,
            pltpu.SemaphoreType.DMA,
            pltpu.SemaphoreType.DMA,
            pltpu.SemaphoreType.DMA,
        ],
        compiler_params=pltpu.CompilerParams(needs_layout_passes=False),
    )
    def sc_gather(uidx_hbm, iidx_hbm, utabT_hbm, itabT_hbm,
                  utail_hbm, itail_hbm,
                  uout_hbm, iout_hbm,
                  idx_v, my_c, my_b, now_c, now_b,
                  slab_a, slab_b, utail_v, itail_v, outstage, outstage2,
                  sem, sem2, sem_a, sem_b):
        wid = lax.axis_index("s") * nc + lax.axis_index("c")
        wid16 = jnp.full((16,), wid, jnp.int32)
        iota = lax.iota(jnp.int32, 16)
        dvecs = [iota + 16 * q for q in range(4)]

        def phase(idx_hbm, tabT_hbm, tail_ref, out_hbm,
                  nslab, main_end, bound):
            pltpu.sync_copy(idx_hbm, idx_v)

            def fbody(i, cnt):
                v = idx_v[pl.ds(i * 16, 16)]
                p = iota + i * 16
                m = (((v >> 9) & (nw - 1)) == wid16) | (v >= main_end)
                mi = m.astype(jnp.int32)
                npop = plsc.all_reduce_population_count(m)[0]

                @pl.when(npop > 0)
                def _():
                    pos = (jnp.full((16,), cnt, jnp.int32)
                           + _excl_prefix(mi, iota))
                    plsc.store_scatter(my_c, [pos], v, mask=m)
                    plsc.store_scatter(my_b, [pos], p, mask=m)

                return cnt + npop

            cnt = lax.fori_loop(0, BATCH // 16, fbody, jnp.int32(0),
                                unroll=False)
            cnt16 = jnp.full((16,), cnt, jnp.int32)
            ngroups_my = (cnt + 15) // 16

            def do_block(block_ref, c0, is_tail, s):
                s16 = jnp.full((16,), s, jnp.int32)

                def rbody(g, cnt2):
                    c = my_c[pl.ds(g * 16, 16)]
                    b = my_b[pl.ds(g * 16, 16)]
                    if is_tail:
                        m2 = c >= main_end
                    else:
                        m2 = (c >> 9) == s16
                    m2 = m2 & ((iota + g * 16) < cnt16)
                    mi2 = m2.astype(jnp.int32)
                    npop2 = plsc.all_reduce_population_count(m2)[0]

                    @pl.when(npop2 > 0)
                    def _():
                        pos2 = (jnp.full((16,), cnt2, jnp.int32)
                                + _excl_prefix(mi2, iota))
                        plsc.store_scatter(now_c, [pos2], c - c0, mask=m2)
                        plsc.store_scatter(now_b, [pos2], b, mask=m2)

                    return cnt2 + npop2

                cnt2 = lax.fori_loop(0, ngroups_my, rbody, jnp.int32(0),
                                     unroll=False)

                @pl.when(cnt2 > 0)
                def _():
                    c0v = now_c[pl.ds(0, 16)]
                    b0v = now_b[pl.ds(0, 16)]
                    now_c[pl.ds(cnt2, 16)] = jnp.full((16,), c0v[0], jnp.int32)
                    now_b[pl.ds(cnt2, 16)] = jnp.full((16,), b0v[0], jnp.int32)

                    ngroups = (cnt2 + 15) // 16

                    def fill(g, stage, sem_x):
                        cvec = now_c[pl.ds(g * 16, 16)]
                        bvec = now_b[pl.ds(g * 16, 16)]
                        for l in range(16):
                            cl16 = jnp.full((16,), cvec[l], jnp.int32)
                            for q in range(4):
                                stage[l, pl.ds(16 * q, 16)] = (
                                    plsc.load_gather(
                                        block_ref, [dvecs[q], cl16]))
                        pltpu.async_copy(stage, out_hbm.at[bvec], sem_x)

                    def drain(stage, sem_x):
                        pltpu.make_async_copy(
                            stage, out_hbm.at[pl.ds(0, 16)], sem_x).wait()

                    def gbody(g, carry):
                        even = (g & 1) == 0

                        @pl.when(even & (g >= 2))
                        def _():
                            drain(outstage, sem)

                        @pl.when(even)
                        def _():
                            fill(g, outstage, sem)

                        @pl.when(jnp.logical_not(even) & (g >= 3))
                        def _():
                            drain(outstage2, sem2)

                        @pl.when(jnp.logical_not(even))
                        def _():
                            fill(g, outstage2, sem2)

                        return carry

                    lax.fori_loop(0, ngroups, gbody, 0, unroll=False)

                    @pl.when(ngroups > 0)
                    def _():
                        drain(outstage, sem)

                    @pl.when(ngroups > 1)
                    def _():
                        drain(outstage2, sem2)

            def issue(s, buf, sem_x):
                @pl.when(s < nslab)
                def _():
                    pltpu.async_copy(
                        tabT_hbm.at[:, pl.ds(s * SLAB, SLAB)], buf, sem_x)

            def wait_proc(s, buf, sem_x):
                @pl.when(s < nslab)
                def _():
                    pltpu.make_async_copy(
                        tabT_hbm.at[:, pl.ds(0, SLAB)], buf, sem_x).wait()
                    do_block(buf, s * SLAB, False, s)

            issue(wid, slab_a, sem_a)
            issue(wid + nw, slab_b, sem_b)

            def sbody(j, carry):
                s0 = wid + nw * (2 * j)
                s1 = s0 + nw
                wait_proc(s0, slab_a, sem_a)
                issue(s0 + 2 * nw, slab_a, sem_a)
                wait_proc(s1, slab_b, sem_b)
                issue(s1 + 2 * nw, slab_b, sem_b)
                return carry

            lax.fori_loop(0, -(-bound // 2), sbody, 0, unroll=False)
            do_block(tail_ref, main_end, True, 0)

        pltpu.sync_copy(utail_hbm, utail_v)
        pltpu.sync_copy(itail_hbm, itail_v)
        phase(uidx_hbm, utabT_hbm, utail_v, uout_hbm,
              u_nslab, u_main, u_bound)
        phase(iidx_hbm, itabT_hbm, itail_v, iout_hbm,
              i_nslab, i_main, i_bound)

    return sc_gather


def _mlp_body(u_ref, i_ref, w1u_ref, w1i_ref, b1_ref,
              w2_ref, b2_ref, w3_ref, b3_ref, out_ref):
    u = u_ref[:, 0:64]
    i = i_ref[:, 0:64]
    h = (jnp.dot(u, w1u_ref[...], preferred_element_type=jnp.float32)
         + jnp.dot(i, w1i_ref[...], preferred_element_type=jnp.float32)
         + b1_ref[...])
    h = jnp.maximum(h, 0.0)
    h2 = jnp.dot(h, w2_ref[...], preferred_element_type=jnp.float32) + b2_ref[...]
    h2 = jnp.maximum(h2, 0.0)
    logits = jnp.sum(h2 * w3_ref[...], axis=1) + b3_ref[0, 0]
    out_ref[...] = jax.nn.sigmoid(logits).reshape(out_ref.shape)


def kernel(user_indices, item_indices, user_table, item_table,
           W1, b1, W2, b2, W3, b3):
    uidx = user_indices.astype(jnp.int32)
    iidx = item_indices.astype(jnp.int32)
    num_users = user_table.shape[0]
    num_items = item_table.shape[0]
    u_main = (num_users // SLAB) * SLAB
    i_main = (num_items // SLAB) * SLAB
    utabT = user_table.T
    itabT = item_table.T
    sc_gather = _make_sc_gather(num_users, num_items)
    u_emb, i_emb = sc_gather(uidx, iidx, utabT, itabT,
                             utabT[:, u_main:], itabT[:, i_main:])

    blk = 2048
    nblk = BATCH // blk
    w1u = W1[:EMBED_DIM]
    w1i = W1[EMBED_DIM:]
    out2d = pl.pallas_call(
        _mlp_body,
        grid=(nblk,),
        in_specs=[
            pl.BlockSpec((blk, 128), lambda i: (i, 0)),
            pl.BlockSpec((blk, 128), lambda i: (i, 0)),
            pl.BlockSpec((EMBED_DIM, 64), lambda i: (0, 0)),
            pl.BlockSpec((EMBED_DIM, 64), lambda i: (0, 0)),
            pl.BlockSpec((1, 64), lambda i: (0, 0)),
            pl.BlockSpec((64, 32), lambda i: (0, 0)),
            pl.BlockSpec((1, 32), lambda i: (0, 0)),
            pl.BlockSpec((1, 32), lambda i: (0, 0)),
            pl.BlockSpec((1, 1), lambda i: (0, 0), memory_space=pltpu.SMEM),
        ],
        out_specs=pl.BlockSpec((1, 8, blk // 8), lambda i: (i, 0, 0)),
        out_shape=jax.ShapeDtypeStruct((nblk, 8, blk // 8), jnp.float32),
    )(u_emb, i_emb, w1u, w1i, b1[None, :], W2, b2[None, :],
      W3.reshape(1, 32), b3.reshape(1, 1))
    return out2d.reshape(BATCH)


# speedup vs baseline: 1.2336x; 1.2336x over previous
import functools

import jax
import jax.numpy as jnp
from jax import lax
from jax.experimental import pallas as pl
from jax.experimental.pallas import tpu as pltpu
from jax.experimental.pallas import tpu_sc as plsc

BATCH = 16384
EMBED_DIM = 64
SLAB = 512
CAP = 4224


def _excl_prefix(mi, iota):
    del iota
    return plsc.cumsum(mi) - mi


def _make_sc_gather(num_users, num_items):
    info = plsc.get_sparse_core_info()
    nc, ns = info.num_cores, info.num_subcores
    nw = nc * ns

    u_nslab = num_users // SLAB
    u_main = u_nslab * SLAB
    u_tail = num_users - u_main
    i_nslab = num_items // SLAB
    i_main = i_nslab * SLAB
    i_tail = num_items - i_main
    u_bound = -(-u_nslab // nw)
    i_bound = -(-i_nslab // nw)

    mesh = plsc.VectorSubcoreMesh(core_axis_name="c", subcore_axis_name="s")

    @functools.partial(
        pl.kernel,
        mesh=mesh,
        out_type=[
            jax.ShapeDtypeStruct((BATCH, 128), jnp.float32),
            jax.ShapeDtypeStruct((BATCH, 128), jnp.float32),
        ],
        scratch_types=[
            pltpu.VMEM((BATCH,), jnp.int32),
            pltpu.VMEM((CAP + 16,), jnp.int32),
            pltpu.VMEM((CAP + 16,), jnp.int32),
            pltpu.VMEM((CAP + 16,), jnp.int32),
            pltpu.VMEM((CAP + 16,), jnp.int32),
            pltpu.VMEM((EMBED_DIM, SLAB), jnp.float32),
            pltpu.VMEM((EMBED_DIM, SLAB), jnp.float32),
            pltpu.VMEM((EMBED_DIM, u_tail), jnp.float32),
            pltpu.VMEM((EMBED_DIM, i_tail), jnp.float32),
            pltpu.VMEM((16, 128), jnp.float32),
            pltpu.VMEM((16, 128), jnp.float32),
            pltpu.SemaphoreType.DMA,
            pltpu.SemaphoreType.DMA,
            pltpu.SemaphoreType.DMA,
            pltpu.SemaphoreType.DMA,
        ],
        compiler_params=pltpu.CompilerParams(needs_layout_passes=False),
    )
    def sc_gather(uidx_hbm, iidx_hbm, utabT_hbm, itabT_hbm,
                  utail_hbm, itail_hbm,
                  uout_hbm, iout_hbm,
                  idx_v, my_c, my_b, now_c, now_b,
                  slab_a, slab_b, utail_v, itail_v, outstage, outstage2,
                  sem, sem2, sem_a, sem_b):
        wid = lax.axis_index("s") * nc + lax.axis_index("c")
        wid16 = jnp.full((16,), wid, jnp.int32)
        iota = lax.iota(jnp.int32, 16)
        dvecs = [iota + 16 * q for q in range(4)]

        def phase(idx_hbm, tabT_hbm, tail_ref, out_hbm,
                  nslab, main_end, bound):
            pltpu.sync_copy(idx_hbm, idx_v)

            def fbody(i, cnt):
                v = idx_v[pl.ds(i * 16, 16)]
                p = iota + i * 16
                m = (((v >> 9) & (nw - 1)) == wid16) | (v >= main_end)
                mi = m.astype(jnp.int32)
                pos = jnp.full((16,), cnt, jnp.int32) + _excl_prefix(mi, iota)
                plsc.store_scatter(my_c, [pos], v, mask=m)
                plsc.store_scatter(my_b, [pos], p, mask=m)
                return cnt + plsc.all_reduce_population_count(m)[0]

            cnt = lax.fori_loop(0, BATCH // 16, fbody, jnp.int32(0),
                                unroll=False)
            cnt16 = jnp.full((16,), cnt, jnp.int32)
            ngroups_my = (cnt + 15) // 16

            def do_block(block_ref, c0, is_tail, s):
                s16 = jnp.full((16,), s, jnp.int32)

                def rbody(g, cnt2):
                    c = my_c[pl.ds(g * 16, 16)]
                    b = my_b[pl.ds(g * 16, 16)]
                    if is_tail:
                        m2 = c >= main_end
                    else:
                        m2 = (c >> 9) == s16
                    m2 = m2 & ((iota + g * 16) < cnt16)
                    mi2 = m2.astype(jnp.int32)
                    pos2 = (jnp.full((16,), cnt2, jnp.int32)
                            + _excl_prefix(mi2, iota))
                    plsc.store_scatter(now_c, [pos2], c - c0, mask=m2)
                    plsc.store_scatter(now_b, [pos2], b, mask=m2)
                    return cnt2 + plsc.all_reduce_population_count(m2)[0]

                cnt2 = lax.fori_loop(0, ngroups_my, rbody, jnp.int32(0),
                                     unroll=False)

                @pl.when(cnt2 > 0)
                def _():
                    c0v = now_c[pl.ds(0, 16)]
                    b0v = now_b[pl.ds(0, 16)]
                    now_c[pl.ds(cnt2, 16)] = jnp.full((16,), c0v[0], jnp.int32)
                    now_b[pl.ds(cnt2, 16)] = jnp.full((16,), b0v[0], jnp.int32)

                    def gbody(g, carry):
                        cvec = now_c[pl.ds(g * 16, 16)]
                        bvec = now_b[pl.ds(g * 16, 16)]
                        for l in range(16):
                            cl16 = jnp.full((16,), cvec[l], jnp.int32)
                            for q in range(4):
                                outstage[l, pl.ds(16 * q, 16)] = (
                                    plsc.load_gather(
                                        block_ref, [dvecs[q], cl16]))
                        pltpu.async_copy(
                            outstage, out_hbm.at[bvec], sem).wait()
                        return carry

                    lax.fori_loop(0, (cnt2 + 15) // 16, gbody, 0,
                                  unroll=False)

            def issue(s, buf, sem_x):
                @pl.when(s < nslab)
                def _():
                    pltpu.async_copy(
                        tabT_hbm.at[:, pl.ds(s * SLAB, SLAB)], buf, sem_x)

            def wait_proc(s, buf, sem_x):
                @pl.when(s < nslab)
                def _():
                    pltpu.make_async_copy(
                        tabT_hbm.at[:, pl.ds(0, SLAB)], buf, sem_x).wait()
                    do_block(buf, s * SLAB, False, s)

            issue(wid, slab_a, sem_a)
            issue(wid + nw, slab_b, sem_b)

            def sbody(j, carry):
                s0 = wid + nw * (2 * j)
                s1 = s0 + nw
                wait_proc(s0, slab_a, sem_a)
                issue(s0 + 2 * nw, slab_a, sem_a)
                wait_proc(s1, slab_b, sem_b)
                issue(s1 + 2 * nw, slab_b, sem_b)
                return carry

            lax.fori_loop(0, -(-bound // 2), sbody, 0, unroll=False)
            do_block(tail_ref, main_end, True, 0)

        pltpu.sync_copy(utail_hbm, utail_v)
        pltpu.sync_copy(itail_hbm, itail_v)
        phase(uidx_hbm, utabT_hbm, utail_v, uout_hbm,
              u_nslab, u_main, u_bound)
        phase(iidx_hbm, itabT_hbm, itail_v, iout_hbm,
              i_nslab, i_main, i_bound)

    return sc_gather


def _mlp_body(u_ref, i_ref, w1u_ref, w1i_ref, b1_ref,
              w2_ref, b2_ref, w3_ref, b3_ref, out_ref):
    u = u_ref[:, 0:64]
    i = i_ref[:, 0:64]
    h = (jnp.dot(u, w1u_ref[...], preferred_element_type=jnp.float32)
         + jnp.dot(i, w1i_ref[...], preferred_element_type=jnp.float32)
         + b1_ref[...])
    h = jnp.maximum(h, 0.0)
    h2 = jnp.dot(h, w2_ref[...], preferred_element_type=jnp.float32) + b2_ref[...]
    h2 = jnp.maximum(h2, 0.0)
    logits = jnp.sum(h2 * w3_ref[...], axis=1) + b3_ref[0, 0]
    out_ref[...] = jax.nn.sigmoid(logits).reshape(out_ref.shape)


def kernel(user_indices, item_indices, user_table, item_table,
           W1, b1, W2, b2, W3, b3):
    uidx = user_indices.astype(jnp.int32)
    iidx = item_indices.astype(jnp.int32)
    num_users = user_table.shape[0]
    num_items = item_table.shape[0]
    u_main = (num_users // SLAB) * SLAB
    i_main = (num_items // SLAB) * SLAB
    utabT = user_table.T
    itabT = item_table.T
    sc_gather = _make_sc_gather(num_users, num_items)
    u_emb, i_emb = sc_gather(uidx, iidx, utabT, itabT,
                             utabT[:, u_main:], itabT[:, i_main:])

    blk = 2048
    nblk = BATCH // blk
    w1u = W1[:EMBED_DIM]
    w1i = W1[EMBED_DIM:]
    out2d = pl.pallas_call(
        _mlp_body,
        grid=(nblk,),
        in_specs=[
            pl.BlockSpec((blk, 128), lambda i: (i, 0)),
            pl.BlockSpec((blk, 128), lambda i: (i, 0)),
            pl.BlockSpec((EMBED_DIM, 64), lambda i: (0, 0)),
            pl.BlockSpec((EMBED_DIM, 64), lambda i: (0, 0)),
            pl.BlockSpec((1, 64), lambda i: (0, 0)),
            pl.BlockSpec((64, 32), lambda i: (0, 0)),
            pl.BlockSpec((1, 32), lambda i: (0, 0)),
            pl.BlockSpec((1, 32), lambda i: (0, 0)),
            pl.BlockSpec((1, 1), lambda i: (0, 0), memory_space=pltpu.SMEM),
        ],
        out_specs=pl.BlockSpec((1, 8, blk // 8), lambda i: (i, 0, 0)),
        out_shape=jax.ShapeDtypeStruct((nblk, 8, blk // 8), jnp.float32),
    )(u_emb, i_emb, w1u, w1i, b1[None, :], W2, b2[None, :],
      W3.reshape(1, 32), b3.reshape(1, 1))
    return out2d.reshape(BATCH)


# speedup vs baseline: 1.2433x; 1.0078x over previous
import functools

import jax
import jax.numpy as jnp
from jax import lax
from jax.experimental import pallas as pl
from jax.experimental.pallas import tpu as pltpu
from jax.experimental.pallas import tpu_sc as plsc

BATCH = 16384
EMBED_DIM = 64
SLAB = 512
CAP = 4224


def _excl_prefix(mi, iota):
    del iota
    return plsc.cumsum(mi) - mi


def _make_sc_gather(num_users, num_items):
    info = plsc.get_sparse_core_info()
    nc, ns = info.num_cores, info.num_subcores
    nw = nc * ns

    u_nslab = num_users // SLAB
    u_main = u_nslab * SLAB
    u_tail = num_users - u_main
    i_nslab = num_items // SLAB
    i_main = i_nslab * SLAB
    i_tail = num_items - i_main
    u_bound = -(-u_nslab // nw)
    i_bound = -(-i_nslab // nw)

    mesh = plsc.VectorSubcoreMesh(core_axis_name="c", subcore_axis_name="s")

    @functools.partial(
        pl.kernel,
        mesh=mesh,
        out_type=[
            jax.ShapeDtypeStruct((BATCH, 128), jnp.float32),
            jax.ShapeDtypeStruct((BATCH, 128), jnp.float32),
        ],
        scratch_types=[
            pltpu.VMEM((BATCH,), jnp.int32),
            pltpu.VMEM((CAP + 16,), jnp.int32),
            pltpu.VMEM((CAP + 16,), jnp.int32),
            pltpu.VMEM((CAP + 16,), jnp.int32),
            pltpu.VMEM((CAP + 16,), jnp.int32),
            pltpu.VMEM((EMBED_DIM, SLAB), jnp.float32),
            pltpu.VMEM((EMBED_DIM, SLAB), jnp.float32),
            pltpu.VMEM((EMBED_DIM, u_tail), jnp.float32),
            pltpu.VMEM((EMBED_DIM, i_tail), jnp.float32),
            pltpu.VMEM((16, 128), jnp.float32),
            pltpu.VMEM((16, 128), jnp.float32),
            pltpu.SemaphoreType.DMA,
            pltpu.SemaphoreType.DMA,
            pltpu.SemaphoreType.DMA,
            pltpu.SemaphoreType.DMA,
        ],
        compiler_params=pltpu.CompilerParams(needs_layout_passes=False),
    )
    def sc_gather(uidx_hbm, iidx_hbm, utabT_hbm, itabT_hbm,
                  utail_hbm, itail_hbm,
                  uout_hbm, iout_hbm,
                  idx_v, my_c, my_b, now_c, now_b,
                  slab_a, slab_b, utail_v, itail_v, outstage, outstage2,
                  sem, sem2, sem_a, sem_b):
        wid = lax.axis_index("s") * nc + lax.axis_index("c")
        wid16 = jnp.full((16,), wid, jnp.int32)
        iota = lax.iota(jnp.int32, 16)
        dvecs = [iota + 16 * q for q in range(4)]

        def phase(idx_hbm, tabT_hbm, tail_ref, out_hbm,
                  nslab, main_end, bound):
            pltpu.sync_copy(idx_hbm, idx_v)

            def fbody(i, cnt):
                v = idx_v[pl.ds(i * 16, 16)]
                p = iota + i * 16
                m = (((v >> 9) & (nw - 1)) == wid16) | (v >= main_end)
                mi = m.astype(jnp.int32)
                pos = jnp.full((16,), cnt, jnp.int32) + _excl_prefix(mi, iota)
                plsc.store_scatter(my_c, [pos], v, mask=m)
                plsc.store_scatter(my_b, [pos], p, mask=m)
                return cnt + plsc.all_reduce_population_count(m)[0]

            cnt = lax.fori_loop(0, BATCH // 16, fbody, jnp.int32(0),
                                unroll=False)
            cnt16 = jnp.full((16,), cnt, jnp.int32)
            ngroups_my = (cnt + 15) // 16

            def do_block(block_ref, c0, is_tail, s):
                s16 = jnp.full((16,), s, jnp.int32)

                def rbody(g, cnt2):
                    c = my_c[pl.ds(g * 16, 16)]
                    b = my_b[pl.ds(g * 16, 16)]
                    if is_tail:
                        m2 = c >= main_end
                    else:
                        m2 = (c >> 9) == s16
                    m2 = m2 & ((iota + g * 16) < cnt16)
                    mi2 = m2.astype(jnp.int32)
                    pos2 = (jnp.full((16,), cnt2, jnp.int32)
                            + _excl_prefix(mi2, iota))
                    plsc.store_scatter(now_c, [pos2], c - c0, mask=m2)
                    plsc.store_scatter(now_b, [pos2], b, mask=m2)
                    return cnt2 + plsc.all_reduce_population_count(m2)[0]

                cnt2 = lax.fori_loop(0, ngroups_my, rbody, jnp.int32(0),
                                     unroll=False)

                @pl.when(cnt2 > 0)
                def _():
                    c0v = now_c[pl.ds(0, 16)]
                    b0v = now_b[pl.ds(0, 16)]
                    now_c[pl.ds(cnt2, 16)] = jnp.full((16,), c0v[0], jnp.int32)
                    now_b[pl.ds(cnt2, 16)] = jnp.full((16,), b0v[0], jnp.int32)

                    def gbody(g, carry):
                        cvec = now_c[pl.ds(g * 16, 16)]
                        bvec = now_b[pl.ds(g * 16, 16)]
                        for l in range(16):
                            cl16 = jnp.full((16,), cvec[l], jnp.int32)
                            for q in range(4):
                                outstage[l, pl.ds(16 * q, 16)] = (
                                    plsc.load_gather(
                                        block_ref, [dvecs[q], cl16]))
                        pltpu.async_copy(
                            outstage, out_hbm.at[bvec], sem).wait()
                        return carry

                    lax.fori_loop(0, (cnt2 + 15) // 16, gbody, 0,
                                  unroll=False)

            def issue(s, buf, sem_x):
                @pl.when(s < nslab)
                def _():
                    pltpu.async_copy(
                        tabT_hbm.at[:, pl.ds(s * SLAB, SLAB)], buf, sem_x)

            def wait_proc(s, buf, sem_x):
                @pl.when(s < nslab)
                def _():
                    pltpu.make_async_copy(
                        tabT_hbm.at[:, pl.ds(0, SLAB)], buf, sem_x).wait()
                    do_block(buf, s * SLAB, False, s)

            issue(wid, slab_a, sem_a)
            issue(wid + nw, slab_b, sem_b)

            def sbody(j, carry):
                s0 = wid + nw * (2 * j)
                s1 = s0 + nw
                wait_proc(s0, slab_a, sem_a)
                issue(s0 + 2 * nw, slab_a, sem_a)
                wait_proc(s1, slab_b, sem_b)
                issue(s1 + 2 * nw, slab_b, sem_b)
                return carry

            lax.fori_loop(0, -(-bound // 2), sbody, 0, unroll=False)
            do_block(tail_ref, main_end, True, 0)

        pltpu.sync_copy(utail_hbm, utail_v)
        pltpu.sync_copy(itail_hbm, itail_v)
        phase(uidx_hbm, utabT_hbm, utail_v, uout_hbm,
              u_nslab, u_main, u_bound)
        phase(iidx_hbm, itabT_hbm, itail_v, iout_hbm,
              i_nslab, i_main, i_bound)

    return sc_gather


def _mlp_body(u_ref, i_ref, w1u_ref, w1i_ref, b1_ref,
              w2_ref, b2_ref, w3_ref, b3_ref, out_ref):
    u = u_ref[:, 0:64]
    i = i_ref[:, 0:64]
    h = (jnp.dot(u, w1u_ref[...], preferred_element_type=jnp.float32)
         + jnp.dot(i, w1i_ref[...], preferred_element_type=jnp.float32)
         + b1_ref[...])
    h = jnp.maximum(h, 0.0)
    h2 = jnp.dot(h, w2_ref[...], preferred_element_type=jnp.float32) + b2_ref[...]
    h2 = jnp.maximum(h2, 0.0)
    logits = jnp.sum(h2 * w3_ref[...], axis=1) + b3_ref[0, 0]
    out_ref[...] = jax.nn.sigmoid(logits).reshape(out_ref.shape)


def kernel(user_indices, item_indices, user_table, item_table,
           W1, b1, W2, b2, W3, b3):
    uidx = user_indices.astype(jnp.int32)
    iidx = item_indices.astype(jnp.int32)
    num_users = user_table.shape[0]
    num_items = item_table.shape[0]
    u_main = (num_users // SLAB) * SLAB
    i_main = (num_items // SLAB) * SLAB
    utabT = user_table.T
    itabT = item_table.T
    sc_gather = _make_sc_gather(num_users, num_items)
    u_emb, i_emb = sc_gather(uidx, iidx, utabT, itabT,
                             utabT[:, u_main:], itabT[:, i_main:])

    blk = 4096
    nblk = BATCH // blk
    w1u = W1[:EMBED_DIM]
    w1i = W1[EMBED_DIM:]
    out2d = pl.pallas_call(
        _mlp_body,
        grid=(nblk,),
        in_specs=[
            pl.BlockSpec((blk, 128), lambda i: (i, 0)),
            pl.BlockSpec((blk, 128), lambda i: (i, 0)),
            pl.BlockSpec((EMBED_DIM, 64), lambda i: (0, 0)),
            pl.BlockSpec((EMBED_DIM, 64), lambda i: (0, 0)),
            pl.BlockSpec((1, 64), lambda i: (0, 0)),
            pl.BlockSpec((64, 32), lambda i: (0, 0)),
            pl.BlockSpec((1, 32), lambda i: (0, 0)),
            pl.BlockSpec((1, 32), lambda i: (0, 0)),
            pl.BlockSpec((1, 1), lambda i: (0, 0), memory_space=pltpu.SMEM),
        ],
        out_specs=pl.BlockSpec((1, 8, blk // 8), lambda i: (i, 0, 0)),
        out_shape=jax.ShapeDtypeStruct((nblk, 8, blk // 8), jnp.float32),
    )(u_emb, i_emb, w1u, w1i, b1[None, :], W2, b2[None, :],
      W3.reshape(1, 32), b3.reshape(1, 1))
    return out2d.reshape(BATCH)
